# baseline (speedup 1.0000x reference)
import functools

import jax
import jax.numpy as jnp
from jax import lax
from jax.experimental import pallas as pl
from jax.experimental.pallas import tpu as pltpu
from jax.experimental.pallas import tpu_sc as plsc

_LAMDA = 0.5
_NC = 2
_NS = 16
_NW = _NC * _NS
_L = 16


def _make_sc_kernel(batch, dim, num_classes):
    rows_per_w = batch // _NW
    chunk = 64
    n_chunks = rows_per_w // chunk
    n_buf = 2
    n_acc = dim // _L

    mesh = plsc.VectorSubcoreMesh(
        core_axis_name="c", subcore_axis_name="s",
        num_cores=_NC, num_subcores=_NS)

    @functools.partial(
        pl.kernel,
        out_type=jax.ShapeDtypeStruct((_NW, _L), jnp.float32),
        mesh=mesh,
        scratch_types=[
            pltpu.VMEM((rows_per_w,), jnp.int32),
            [pltpu.VMEM((chunk, dim), jnp.float32)] * n_buf,
            [pltpu.VMEM((chunk, dim), jnp.float32)] * n_buf,
            pltpu.VMEM((_L,), jnp.float32),
            [pltpu.SemaphoreType.DMA] * n_buf,
            [pltpu.SemaphoreType.DMA] * n_buf,
        ],
    )
    def sck(feat_hbm, tgt_hbm, cent_hbm, out_hbm, idx_v, gbufs, fbufs,
            acc_v, gsems, fsems):
        wid = lax.axis_index("s") * _NC + lax.axis_index("c")
        base = wid * rows_per_w

        def start_chunk(c, b):
            pltpu.async_copy(
                cent_hbm.at[idx_v.at[pl.ds(c * chunk, chunk)]],
                gbufs[b], gsems[b])
            pltpu.async_copy(
                feat_hbm.at[pl.ds(base + c * chunk, chunk)],
                fbufs[b], fsems[b])

        def wait_chunk(b):
            pltpu.make_async_copy(
                cent_hbm.at[idx_v.at[pl.ds(0, chunk)]],
                gbufs[b], gsems[b]).wait()
            pltpu.make_async_copy(
                feat_hbm.at[pl.ds(0, chunk)], fbufs[b], fsems[b]).wait()

        pltpu.sync_copy(tgt_hbm.at[pl.ds(base, chunk)],
                        idx_v.at[pl.ds(0, chunk)])
        start_chunk(0, 0)
        pltpu.sync_copy(tgt_hbm.at[pl.ds(base + chunk, rows_per_w - chunk)],
                        idx_v.at[pl.ds(chunk, rows_per_w - chunk)])
        start_chunk(1, 1)

        accs0 = tuple(jnp.zeros((_L,), jnp.float32) for _ in range(n_acc))

        def pair_body(c2, accs):
            for b in range(n_buf):
                c = c2 * n_buf + b
                wait_chunk(b)
                gbuf = gbufs[b]
                fbuf = fbufs[b]

                @plsc.parallel_loop(0, chunk, carry=accs, unroll=2)
                def accs(i, a):
                    out = []
                    for k in range(n_acc):
                        d = (gbuf[i, pl.ds(k * _L, _L)]
                             - fbuf[i, pl.ds(k * _L, _L)])
                        out.append(a[k] + d * d)
                    return tuple(out)

                @pl.when(c + n_buf < n_chunks)
                def _():
                    start_chunk(c + n_buf, b)
            return accs

        accs = lax.fori_loop(0, n_chunks // n_buf, pair_body, accs0)

        total = accs[0]
        for k in range(1, n_acc):
            total = total + accs[k]
        acc_v[...] = total
        pltpu.sync_copy(acc_v, out_hbm.at[wid])

    return sck


def kernel(feat, target, centers):
    batch, dim = feat.shape
    sck = _make_sc_kernel(batch, dim, centers.shape[0])
    partials = sck(feat, target.astype(jnp.int32), centers)
    return _LAMDA * jnp.sum(partials) / 2.0 / batch
